# baseline (speedup 1.0000x reference)
import jax
import jax.numpy as jnp
from jax import lax
from jax.experimental import pallas as pl
from jax.experimental.pallas import tpu as pltpu
from jax.experimental.pallas import tpu_sc as plsc

_T = 26
_V = 100000
_D = 64
_B = 4096
_NC = 2
_NS = 16
_NW = _NC * _NS


_VB = (_V + 127) // 128
_VP2 = _VB * 64
_UNITS = _T * _VB


def _tbody(tabT_hbm, pairs_hbm, in_v, out_v, isem, osem):
    wid = lax.axis_index("s") * _NC + lax.axis_index("c")
    iota = lax.iota(jnp.int32, 16)

    nu = (_UNITS - wid + _NW - 1) // _NW

    def unit_tc(u):
        t = u // _VB
        c = u - t * _VB
        return t, c

    def start_in(u, slot):
        t, c = unit_tc(u)
        pltpu.async_copy(
            tabT_hbm.at[t, :, pl.ds(c * 128, 128)], in_v.at[slot], isem)

    start_in(wid, 0)

    def shuffle(slot, nq):
        def q_iter(q, _):
            col = 2 * q
            for h in range(2):
                cvec = lax.broadcast(col + h, (16,))
                for d0 in range(0, 64, 16):
                    vals = plsc.load_gather(
                        in_v.at[slot], [iota + d0, cvec])
                    out_v[slot, q, pl.ds(h * 64 + d0, 16)] = vals
            return 0

        lax.fori_loop(0, nq, q_iter, 0)

    def unit(k, _):
        u = wid + k * _NW
        slot = k & 1

        @pl.when(k + 1 < nu)
        def _():
            start_in(u + _NW, 1 - slot)

        pltpu.make_async_copy(
            tabT_hbm.at[0, :, pl.ds(0, 128)], in_v.at[slot], isem).wait()

        @pl.when(k >= 2)
        def _():
            pltpu.make_async_copy(
                out_v.at[slot], pairs_hbm.at[0, pl.ds(0, 64), :], osem
            ).wait()

        shuffle(slot, 64)

        t, c = unit_tc(u)
        pltpu.async_copy(
            out_v.at[slot], pairs_hbm.at[t, pl.ds(c * 64, 64), :], osem)
        return 0

    lax.fori_loop(0, nu, unit, 0)

    pltpu.make_async_copy(
        out_v.at[0], pairs_hbm.at[0, pl.ds(0, 64), :], osem).wait()
    pltpu.make_async_copy(
        out_v.at[0], pairs_hbm.at[0, pl.ds(0, 64), :], osem).wait()


_CHUNK = _B // _NW
_K = 4
_G = 3


def _gbody(idx_hbm, pairs_hbm, out_hbm, idx_v, pidx_v, rows_v, out_v,
           gsem, wsem):
    wid = lax.axis_index("s") * _NC + lax.axis_index("c")
    iota = lax.iota(jnp.int32, 16)
    b0 = wid * _CHUNK
    pltpu.sync_copy(idx_hbm.at[:, pl.ds(b0, _CHUNK)], idx_v)

    def prep(t):
        def pr(i, _):
            v = idx_v[t, pl.ds(i * 16, 16)]
            pidx_v[t & (_K - 1), pl.ds(i * 16, 16)] = (
                lax.shift_right_logical(v, 1))
            return 0

        lax.fori_loop(0, _CHUNK // 16, pr, 0)
        pltpu.async_copy(
            pairs_hbm.at[t].at[pidx_v.at[t & (_K - 1)]],
            rows_v.at[t & (_K - 1)], gsem)

    for t in range(_G):
        prep(t)

    def ch(t, _):
        s = t & (_K - 1)
        pltpu.make_async_copy(
            pairs_hbm.at[0].at[pidx_v.at[s]], rows_v.at[s], gsem).wait()

        @pl.when(t + _G < _T)
        def _():
            prep(t + _G)

        oslot = t & 1

        @pl.when(t >= 2)
        def _():
            pltpu.make_async_copy(
                out_v.at[oslot], out_hbm.at[0, pl.ds(0, _CHUNK), :],
                wsem).wait()

        hbase = [(idx_v[t, pl.ds(i0, 16)] & 1) * 64
                 for i0 in range(0, _CHUNK, 16)]

        def sel(d, _):
            for j in range(_CHUNK // 16):
                rvec = iota + j * 16
                vals = plsc.load_gather(
                    rows_v.at[s], [rvec, hbase[j] + d])
                plsc.store_scatter(
                    out_v.at[oslot], [rvec, lax.broadcast(d, (16,))], vals)
            return 0

        lax.fori_loop(0, _D, sel, 0)

        pltpu.async_copy(
            out_v.at[oslot], out_hbm.at[t, pl.ds(b0, _CHUNK), :], wsem)
        return 0

    lax.fori_loop(0, _T, ch, 0)

    pltpu.make_async_copy(
        out_v.at[0], out_hbm.at[0, pl.ds(0, _CHUNK), :], wsem).wait()
    pltpu.make_async_copy(
        out_v.at[0], out_hbm.at[0, pl.ds(0, _CHUNK), :], wsem).wait()


_mesh = plsc.VectorSubcoreMesh(core_axis_name="c", subcore_axis_name="s")

_transpose = pl.kernel(
    _tbody,
    out_type=jax.ShapeDtypeStruct((_T, _VP2, 2 * _D), jnp.float32),
    mesh=_mesh,
    scratch_types=[
        pltpu.VMEM((2, _D, 128), jnp.float32),
        pltpu.VMEM((2, _D, 128), jnp.float32),
        pltpu.SemaphoreType.DMA,
        pltpu.SemaphoreType.DMA,
    ],
    compiler_params=pltpu.CompilerParams(use_tc_tiling_on_sc=True, needs_layout_passes=False),
)

_gather = pl.kernel(
    _gbody,
    out_type=jax.ShapeDtypeStruct((_T, _B, _D), jnp.float32),
    mesh=_mesh,
    scratch_types=[
        pltpu.VMEM((_T, _CHUNK), jnp.int32),
        pltpu.VMEM((_K, _CHUNK), jnp.int32),
        pltpu.VMEM((_K, _CHUNK, 2 * _D), jnp.float32),
        pltpu.VMEM((2, _CHUNK, _D), jnp.float32),
        pltpu.SemaphoreType.DMA,
        pltpu.SemaphoreType.DMA,
    ],
    compiler_params=pltpu.CompilerParams(use_tc_tiling_on_sc=True, needs_layout_passes=False),
)


@jax.jit
def kernel(inputs, tables):
    tabT = jnp.transpose(tables, (0, 2, 1))
    idx = jnp.transpose(inputs).astype(jnp.int32)
    pairs = _transpose(tabT)
    out = _gather(idx, pairs)
    return out.transpose(1, 0, 2)
